# baseline (speedup 1.0000x reference)
import functools

import jax
import jax.numpy as jnp
from jax import lax
from jax.experimental import pallas as pl
from jax.experimental.pallas import tpu as pltpu
from jax.experimental.pallas import tpu_sc as plsc

_EPS = 0.3
_NC = 2
_NS = 16
_LANES = 16
_CHUNK = 128
_DCHUNK = 2048
_IDXW = 128


def _rsqrt_newton(x):
    xi = lax.bitcast_convert_type(x, jnp.int32)
    yi = jnp.int32(0x5F3759DF) - (xi >> 1)
    y = lax.bitcast_convert_type(yi, jnp.float32)
    for _ in range(3):
        y = y * (1.5 - 0.5 * x * y * y)
    return y


def _tanh_via_exp(x):
    t = jnp.exp(-2.0 * jnp.abs(x))
    th = (1.0 - t) / (1.0 + t)
    return jnp.where(x < 0.0, -th, th)


def _make_sc_kernel(N, D, NPAD, n0, n1):
    RPT = NPAD // _NS
    T = n0 + n1
    DSUB = _DCHUNK // _IDXW
    nd16 = T // DSUB
    dtail = T % DSUB
    G = _CHUNK // _LANES

    mesh = plsc.VectorSubcoreMesh(
        core_axis_name="c", subcore_axis_name="s",
        num_cores=_NC, num_subcores=_NS)

    @functools.partial(
        pl.kernel,
        out_type=jax.ShapeDtypeStruct((_NC, NPAD, D), jnp.float32),
        mesh=mesh,
        compiler_params=pltpu.CompilerParams(needs_layout_passes=False),
        scratch_types=[
            pltpu.VMEM_SHARED((NPAD,), jnp.float32),
            pltpu.VMEM_SHARED((NPAD,), jnp.float32),
            pltpu.VMEM_SHARED((NPAD,), jnp.float32),
            pltpu.VMEM_SHARED((NPAD,), jnp.float32),
            pltpu.VMEM_SHARED((NPAD, D), jnp.float32),
            pltpu.VMEM((2, _CHUNK), jnp.int32),
            pltpu.VMEM((2, _CHUNK), jnp.int32),
            pltpu.VMEM((2, _CHUNK + _LANES), jnp.float32),
            pltpu.VMEM((2, _CHUNK), jnp.float32),
            pltpu.VMEM((2, _CHUNK), jnp.float32),
            pltpu.VMEM((2, _CHUNK), jnp.float32),
            pltpu.VMEM((2, _CHUNK), jnp.float32),
            pltpu.VMEM((_DCHUNK,), jnp.float32),
            pltpu.VMEM((DSUB, _IDXW), jnp.int32),
            pltpu.VMEM((RPT,), jnp.float32),
            pltpu.VMEM((2, _CHUNK, D), jnp.float32),
            pltpu.SemaphoreType.DMA,
            pltpu.SemaphoreType.DMA,
        ],
    )
    def sc_kernel(h_hbm, src_hbm, dst_hbm, a1_hbm, a2_hbm, mm_hbm,
                  deg_sh, a1_sh, a2_sh, d_sh, m_sh, src_v, dst_v, e_v,
                  a1g_v, a2g_v, ddg_v, dsg_v, ones_v, didx_v, z_v,
                  rows_v, sem0, sem1):
        sems = (sem0, sem1)
        cid = lax.axis_index("c")
        sid = lax.axis_index("s")
        zeros = jnp.zeros((_LANES,), jnp.float32)
        ones = jnp.ones((_LANES,), jnp.float32)

        @pl.loop(0, RPT // _LANES)
        def _(i):
            z_v[pl.ds(i * _LANES, _LANES)] = zeros

        @pl.loop(0, _DCHUNK // _LANES)
        def _(i):
            ones_v[pl.ds(i * _LANES, _LANES)] = ones

        @pl.loop(0, _IDXW)
        def _(i):
            for k in range(D // _LANES):
                rows_v[0, i, pl.ds(k * _LANES, _LANES)] = zeros

        base = sid * RPT
        pltpu.sync_copy(z_v, deg_sh.at[pl.ds(base, RPT)])
        for r in range(RPT // _IDXW):
            pltpu.sync_copy(rows_v.at[0],
                            m_sh.at[pl.ds(base + r * _IDXW, _IDXW), :])
        pltpu.sync_copy(a1_hbm.at[pl.ds(base, RPT)], a1_sh.at[pl.ds(base, RPT)])
        pltpu.sync_copy(a2_hbm.at[pl.ds(base, RPT)], a2_sh.at[pl.ds(base, RPT)])
        plsc.subcore_barrier()

        drow0 = sid * T

        @pl.loop(0, nd16)
        def _(k):
            row = drow0 + k * DSUB
            pltpu.sync_copy(dst_hbm.at[pl.ds(row, DSUB), :], didx_v)
            for j in range(DSUB):
                pltpu.sync_copy(ones_v.at[pl.ds(j * _IDXW, _IDXW)],
                                deg_sh.at[didx_v.at[j]], add=True)
        if dtail:
            trow = drow0 + nd16 * DSUB
            pltpu.sync_copy(dst_hbm.at[pl.ds(trow, dtail), :],
                            didx_v.at[pl.ds(0, dtail), :])
            for j in range(dtail):
                pltpu.sync_copy(ones_v.at[pl.ds(j * _IDXW, _IDXW)],
                                deg_sh.at[didx_v.at[j]], add=True)
        plsc.subcore_barrier()

        pltpu.sync_copy(deg_sh.at[pl.ds(base, RPT)], z_v)

        @pl.loop(0, RPT // _LANES)
        def _(i):
            idx = lax.iota(jnp.int32, _LANES) + (base + i * _LANES)
            x = jnp.maximum(z_v[pl.ds(i * _LANES, _LANES)], 1.0)
            y = _rsqrt_newton(x)
            z_v[pl.ds(i * _LANES, _LANES)] = jnp.where(idx >= N, 0.0, y)

        pltpu.sync_copy(z_v, d_sh.at[pl.ds(base, RPT)])
        plsc.subcore_barrier()

        n_c = jnp.where(cid == 0, n0, n1)
        erow0 = jnp.where(cid == 0, sid * n0, _NS * n0 + sid * n1)

        def _prefetch(krow, nb):
            pltpu.sync_copy(src_hbm.at[krow], src_v.at[nb])
            pltpu.sync_copy(dst_hbm.at[krow], dst_v.at[nb])
            pltpu.async_copy(h_hbm.at[src_v.at[nb]], rows_v.at[nb], sems[nb])
            pltpu.sync_copy(a1_sh.at[dst_v.at[nb]], a1g_v.at[nb])
            pltpu.sync_copy(a2_sh.at[src_v.at[nb]], a2g_v.at[nb])
            pltpu.sync_copy(d_sh.at[dst_v.at[nb]], ddg_v.at[nb])
            pltpu.sync_copy(d_sh.at[src_v.at[nb]], dsg_v.at[nb])
            for g in range(G):
                sl = pl.ds(g * _LANES, _LANES)
                e_v[nb, sl] = (_tanh_via_exp(a1g_v[nb, sl] + a2g_v[nb, sl])
                               * ddg_v[nb, sl] * dsg_v[nb, sl])

        _prefetch(erow0, 0)

        @pl.loop(0, n_c // 2)
        def _(p):
            for b in range(2):
                k = p * 2 + b
                nb = 1 - b
                nk = jnp.minimum(k + 1, n_c - 1)
                _prefetch(erow0 + nk, nb)
                pltpu.make_async_copy(h_hbm.at[src_v.at[b]],
                                      rows_v.at[b], sems[b]).wait()

                @plsc.parallel_loop(0, _CHUNK, unroll=4)
                def _(i):
                    es = e_v[b, pl.ds(i, _LANES)][0]
                    for kk in range(D // _LANES):
                        rows_v[b, i, pl.ds(kk * _LANES, _LANES)] = (
                            rows_v[b, i, pl.ds(kk * _LANES, _LANES)] * es)

                pltpu.sync_copy(rows_v.at[b], m_sh.at[dst_v.at[b]], add=True)

        pltpu.make_async_copy(h_hbm.at[src_v.at[0]], rows_v.at[0],
                              sems[0]).wait()
        plsc.subcore_barrier()

        for r in range(RPT // _IDXW):
            pltpu.sync_copy(m_sh.at[pl.ds(base + r * _IDXW, _IDXW), :],
                            mm_hbm.at[cid, pl.ds(base + r * _IDXW, _IDXW), :])

    return sc_kernel


def _pick_bs(n):
    for cand in (1024, 1000, 512, 500, 256, 250, 128, 125, 64, 40, 32, 25,
                 16, 10, 8, 5, 4, 2, 1):
        if n % cand == 0:
            return cand
    return 1


def _gate_proj(h, w2, b2):
    n, d = h.shape
    bs = _pick_bs(n)

    def body(h_ref, w_ref, b_ref, o_ref):
        o_ref[...] = jnp.dot(h_ref[...], w_ref[...],
                             preferred_element_type=jnp.float32) + b_ref[...]

    return pl.pallas_call(
        body,
        grid=(n // bs,),
        in_specs=[pl.BlockSpec((bs, d), lambda i: (i, 0)),
                  pl.BlockSpec((d, 2), lambda i: (0, 0)),
                  pl.BlockSpec((1, 2), lambda i: (0, 0))],
        out_specs=pl.BlockSpec((bs, 2), lambda i: (i, 0)),
        out_shape=jax.ShapeDtypeStruct((n, 2), jnp.float32),
    )(h, w2, b2)


def _combine(h, mm):
    n, d = h.shape
    bs = _pick_bs(n)

    def body(h_ref, m0_ref, m1_ref, o_ref):
        o_ref[...] = jnp.maximum(
            _EPS * h_ref[...] + m0_ref[0] + m1_ref[0], 0.0)

    return pl.pallas_call(
        body,
        grid=(n // bs,),
        in_specs=[pl.BlockSpec((bs, d), lambda i: (i, 0)),
                  pl.BlockSpec((1, bs, d), lambda i: (0, i, 0)),
                  pl.BlockSpec((1, bs, d), lambda i: (1, i, 0))],
        out_specs=pl.BlockSpec((bs, d), lambda i: (i, 0)),
        out_shape=jax.ShapeDtypeStruct((n, d), jnp.float32),
    )(h, mm, mm)


_SC0_FRAC = 0.29


def kernel(h, edge_index, gate_w, gate_b):
    n, d = h.shape
    e = edge_index.shape[1]

    npad = -((n + 1) // -(_NS * _LANES)) * (_NS * _LANES)
    per_chunk = _NS * _CHUNK
    t = -(e // -(per_chunk * 16)) * 16
    n0 = int(round(t * _SC0_FRAC / 2)) * 2
    n0 = min(max(n0, 2), t - 2)
    n1 = t - n0
    e_pad = t * per_chunk

    src = edge_index[0]
    dst = edge_index[1]
    pad = e_pad - e
    srcp = jnp.concatenate(
        [src, jnp.zeros((pad,), jnp.int32)]).reshape(e_pad // _IDXW, _IDXW)
    dstp = jnp.concatenate(
        [dst, jnp.full((pad,), n, jnp.int32)]).reshape(e_pad // _IDXW, _IDXW)

    w_dst = gate_w[:d, 0]
    w_src = gate_w[d:, 0]
    w2 = jnp.stack([w_dst, w_src], axis=1)
    b2 = jnp.stack([gate_b[0], jnp.zeros((), jnp.float32)]).reshape(1, 2)

    a = _gate_proj(h, w2, b2)
    a1 = jnp.pad(a[:, 0], (0, npad - n))
    a2 = jnp.pad(a[:, 1], (0, npad - n))

    mm = _make_sc_kernel(n, d, npad, n0, n1)(h, srcp, dstp, a1, a2)
    return _combine(h, mm)


# speedup vs baseline: 1.2675x; 1.2675x over previous
import functools

import jax
import jax.numpy as jnp
from jax import lax
from jax.experimental import pallas as pl
from jax.experimental.pallas import tpu as pltpu
from jax.experimental.pallas import tpu_sc as plsc

_EPS = 0.3
_NC = 2
_NS = 16
_LANES = 16
_CHUNK = 128
_DCHUNK = 2048
_IDXW = 128


def _rsqrt_newton(x):
    xi = lax.bitcast_convert_type(x, jnp.int32)
    yi = jnp.int32(0x5F3759DF) - (xi >> 1)
    y = lax.bitcast_convert_type(yi, jnp.float32)
    for _ in range(3):
        y = y * (1.5 - 0.5 * x * y * y)
    return y


def _tanh_via_exp(x):
    t = jnp.exp(-2.0 * jnp.abs(x))
    th = (1.0 - t) / (1.0 + t)
    return jnp.where(x < 0.0, -th, th)


def _make_sc_kernel(N, D, NPAD, n0, n1):
    RPT = NPAD // _NS
    T = n0 + n1
    DSUB = _DCHUNK // _IDXW
    nd16 = T // DSUB
    dtail = T % DSUB
    G = _CHUNK // _LANES

    mesh = plsc.VectorSubcoreMesh(
        core_axis_name="c", subcore_axis_name="s",
        num_cores=_NC, num_subcores=_NS)

    @functools.partial(
        pl.kernel,
        out_type=jax.ShapeDtypeStruct((_NC, NPAD, D), jnp.float32),
        mesh=mesh,
        compiler_params=pltpu.CompilerParams(needs_layout_passes=False),
        scratch_types=[
            pltpu.VMEM_SHARED((NPAD,), jnp.float32),
            pltpu.VMEM_SHARED((NPAD,), jnp.float32),
            pltpu.VMEM_SHARED((NPAD,), jnp.float32),
            pltpu.VMEM_SHARED((NPAD,), jnp.float32),
            pltpu.VMEM_SHARED((NPAD, D), jnp.float32),
            pltpu.VMEM((2, _CHUNK), jnp.int32),
            pltpu.VMEM((2, _CHUNK), jnp.int32),
            pltpu.VMEM((2, _CHUNK + _LANES), jnp.float32),
            pltpu.VMEM((2, _CHUNK), jnp.float32),
            pltpu.VMEM((2, _CHUNK), jnp.float32),
            pltpu.VMEM((2, _CHUNK), jnp.float32),
            pltpu.VMEM((2, _CHUNK), jnp.float32),
            pltpu.VMEM((_DCHUNK,), jnp.float32),
            pltpu.VMEM((DSUB, _IDXW), jnp.int32),
            pltpu.VMEM((RPT,), jnp.float32),
            pltpu.VMEM((2, _CHUNK, D), jnp.float32),
            pltpu.SemaphoreType.DMA,
            pltpu.SemaphoreType.DMA,
        ],
    )
    def sc_kernel(h_hbm, src_hbm, dst_hbm, a1_hbm, a2_hbm, mm_hbm,
                  deg_sh, a1_sh, a2_sh, d_sh, m_sh, src_v, dst_v, e_v,
                  a1g_v, a2g_v, ddg_v, dsg_v, ones_v, didx_v, z_v,
                  rows_v, sem0, sem1):
        sems = (sem0, sem1)
        cid = lax.axis_index("c")
        sid = lax.axis_index("s")
        zeros = jnp.zeros((_LANES,), jnp.float32)
        ones = jnp.ones((_LANES,), jnp.float32)

        @pl.loop(0, RPT // _LANES)
        def _(i):
            z_v[pl.ds(i * _LANES, _LANES)] = zeros

        @pl.loop(0, _DCHUNK // _LANES)
        def _(i):
            ones_v[pl.ds(i * _LANES, _LANES)] = ones

        @pl.loop(0, _IDXW)
        def _(i):
            for k in range(D // _LANES):
                rows_v[0, i, pl.ds(k * _LANES, _LANES)] = zeros

        base = sid * RPT
        pltpu.sync_copy(z_v, deg_sh.at[pl.ds(base, RPT)])
        for r in range(RPT // _IDXW):
            pltpu.sync_copy(rows_v.at[0],
                            m_sh.at[pl.ds(base + r * _IDXW, _IDXW), :])
        pltpu.sync_copy(a1_hbm.at[pl.ds(base, RPT)], a1_sh.at[pl.ds(base, RPT)])
        pltpu.sync_copy(a2_hbm.at[pl.ds(base, RPT)], a2_sh.at[pl.ds(base, RPT)])
        plsc.subcore_barrier()

        drow0 = sid * T

        @pl.loop(0, nd16)
        def _(k):
            row = drow0 + k * DSUB
            pltpu.sync_copy(dst_hbm.at[pl.ds(row, DSUB), :], didx_v)
            for j in range(DSUB):
                pltpu.sync_copy(ones_v.at[pl.ds(j * _IDXW, _IDXW)],
                                deg_sh.at[didx_v.at[j]], add=True)
        if dtail:
            trow = drow0 + nd16 * DSUB
            pltpu.sync_copy(dst_hbm.at[pl.ds(trow, dtail), :],
                            didx_v.at[pl.ds(0, dtail), :])
            for j in range(dtail):
                pltpu.sync_copy(ones_v.at[pl.ds(j * _IDXW, _IDXW)],
                                deg_sh.at[didx_v.at[j]], add=True)
        plsc.subcore_barrier()

        pltpu.sync_copy(deg_sh.at[pl.ds(base, RPT)], z_v)

        @pl.loop(0, RPT // _LANES)
        def _(i):
            idx = lax.iota(jnp.int32, _LANES) + (base + i * _LANES)
            x = jnp.maximum(z_v[pl.ds(i * _LANES, _LANES)], 1.0)
            y = _rsqrt_newton(x)
            z_v[pl.ds(i * _LANES, _LANES)] = jnp.where(idx >= N, 0.0, y)

        pltpu.sync_copy(z_v, d_sh.at[pl.ds(base, RPT)])
        plsc.subcore_barrier()

        n_c = jnp.where(cid == 0, n0, n1)
        erow0 = jnp.where(cid == 0, sid * n0, _NS * n0 + sid * n1)

        def _prefetch(krow, nb):
            pltpu.sync_copy(src_hbm.at[krow], src_v.at[nb])
            pltpu.sync_copy(dst_hbm.at[krow], dst_v.at[nb])
            pltpu.async_copy(h_hbm.at[src_v.at[nb]], rows_v.at[nb], sems[nb])
            pltpu.sync_copy(a1_sh.at[dst_v.at[nb]], a1g_v.at[nb])
            pltpu.sync_copy(a2_sh.at[src_v.at[nb]], a2g_v.at[nb])
            pltpu.sync_copy(d_sh.at[dst_v.at[nb]], ddg_v.at[nb])
            pltpu.sync_copy(d_sh.at[src_v.at[nb]], dsg_v.at[nb])
            for g in range(G):
                sl = pl.ds(g * _LANES, _LANES)
                e_v[nb, sl] = (_tanh_via_exp(a1g_v[nb, sl] + a2g_v[nb, sl])
                               * ddg_v[nb, sl] * dsg_v[nb, sl])

        _prefetch(erow0, 0)

        @pl.loop(0, n_c // 2)
        def _(p):
            for b in range(2):
                k = p * 2 + b
                nb = 1 - b
                nk = jnp.minimum(k + 1, n_c - 1)
                _prefetch(erow0 + nk, nb)
                pltpu.make_async_copy(h_hbm.at[src_v.at[b]],
                                      rows_v.at[b], sems[b]).wait()

                @plsc.parallel_loop(0, _CHUNK, unroll=4)
                def _(i):
                    es = e_v[b, pl.ds(i, _LANES)][0]
                    for kk in range(D // _LANES):
                        rows_v[b, i, pl.ds(kk * _LANES, _LANES)] = (
                            rows_v[b, i, pl.ds(kk * _LANES, _LANES)] * es)

                pltpu.sync_copy(rows_v.at[b], m_sh.at[dst_v.at[b]], add=True)

        pltpu.make_async_copy(h_hbm.at[src_v.at[0]], rows_v.at[0],
                              sems[0]).wait()
        plsc.subcore_barrier()

        for r in range(RPT // _IDXW):
            pltpu.sync_copy(m_sh.at[pl.ds(base + r * _IDXW, _IDXW), :],
                            mm_hbm.at[cid, pl.ds(base + r * _IDXW, _IDXW), :])

    return sc_kernel


def _pick_bs(n):
    for cand in (1024, 1000, 512, 500, 256, 250, 128, 125, 64, 40, 32, 25,
                 16, 10, 8, 5, 4, 2, 1):
        if n % cand == 0:
            return cand
    return 1


def _gate_proj(h, w2, b2):
    n, d = h.shape
    bs = _pick_bs(n)

    def body(h_ref, w_ref, b_ref, o_ref):
        o_ref[...] = jnp.dot(h_ref[...], w_ref[...],
                             preferred_element_type=jnp.float32) + b_ref[...]

    return pl.pallas_call(
        body,
        grid=(n // bs,),
        in_specs=[pl.BlockSpec((bs, d), lambda i: (i, 0)),
                  pl.BlockSpec((d, 2), lambda i: (0, 0)),
                  pl.BlockSpec((1, 2), lambda i: (0, 0))],
        out_specs=pl.BlockSpec((bs, 2), lambda i: (i, 0)),
        out_shape=jax.ShapeDtypeStruct((n, 2), jnp.float32),
    )(h, w2, b2)


def _combine(h, mm):
    n, d = h.shape
    bs = _pick_bs(n)

    def body(h_ref, m0_ref, m1_ref, o_ref):
        o_ref[...] = jnp.maximum(
            _EPS * h_ref[...] + m0_ref[0] + m1_ref[0], 0.0)

    return pl.pallas_call(
        body,
        grid=(n // bs,),
        in_specs=[pl.BlockSpec((bs, d), lambda i: (i, 0)),
                  pl.BlockSpec((1, bs, d), lambda i: (0, i, 0)),
                  pl.BlockSpec((1, bs, d), lambda i: (1, i, 0))],
        out_specs=pl.BlockSpec((bs, d), lambda i: (i, 0)),
        out_shape=jax.ShapeDtypeStruct((n, d), jnp.float32),
    )(h, mm, mm)


_SC0_FRAC = 0.71


def kernel(h, edge_index, gate_w, gate_b):
    n, d = h.shape
    e = edge_index.shape[1]

    npad = -((n + 1) // -(_NS * _LANES)) * (_NS * _LANES)
    per_chunk = _NS * _CHUNK
    t = -(e // -(per_chunk * 16)) * 16
    n0 = int(round(t * _SC0_FRAC / 2)) * 2
    n0 = min(max(n0, 2), t - 2)
    n1 = t - n0
    e_pad = t * per_chunk

    src = edge_index[0]
    dst = edge_index[1]
    pad = e_pad - e
    srcp = jnp.concatenate(
        [src, jnp.zeros((pad,), jnp.int32)]).reshape(e_pad // _IDXW, _IDXW)
    dstp = jnp.concatenate(
        [dst, jnp.full((pad,), n, jnp.int32)]).reshape(e_pad // _IDXW, _IDXW)

    w_dst = gate_w[:d, 0]
    w_src = gate_w[d:, 0]
    w2 = jnp.stack([w_dst, w_src], axis=1)
    b2 = jnp.stack([gate_b[0], jnp.zeros((), jnp.float32)]).reshape(1, 2)

    a = _gate_proj(h, w2, b2)
    a1 = jnp.pad(a[:, 0], (0, npad - n))
    a2 = jnp.pad(a[:, 1], (0, npad - n))

    mm = _make_sc_kernel(n, d, npad, n0, n1)(h, srcp, dstp, a1, a2)
    return _combine(h, mm)


# speedup vs baseline: 1.2677x; 1.0002x over previous
import functools

import jax
import jax.numpy as jnp
from jax import lax
from jax.experimental import pallas as pl
from jax.experimental.pallas import tpu as pltpu
from jax.experimental.pallas import tpu_sc as plsc

_EPS = 0.3
_NC = 2
_NS = 16
_LANES = 16
_CHUNK = 128
_DCHUNK = 2048
_IDXW = 128


def _rsqrt_newton(x):
    xi = lax.bitcast_convert_type(x, jnp.int32)
    yi = jnp.int32(0x5F3759DF) - (xi >> 1)
    y = lax.bitcast_convert_type(yi, jnp.float32)
    for _ in range(3):
        y = y * (1.5 - 0.5 * x * y * y)
    return y


def _tanh_via_exp(x):
    t = jnp.exp(-2.0 * jnp.abs(x))
    th = (1.0 - t) / (1.0 + t)
    return jnp.where(x < 0.0, -th, th)


def _make_sc_kernel(N, D, NPAD, n0, n1):
    RPT = NPAD // _NS
    T = n0 + n1
    DSUB = _DCHUNK // _IDXW
    nd16 = T // DSUB
    dtail = T % DSUB
    G = _CHUNK // _LANES

    mesh = plsc.VectorSubcoreMesh(
        core_axis_name="c", subcore_axis_name="s",
        num_cores=_NC, num_subcores=_NS)

    @functools.partial(
        pl.kernel,
        out_type=jax.ShapeDtypeStruct((_NC, NPAD, D), jnp.float32),
        mesh=mesh,
        compiler_params=pltpu.CompilerParams(needs_layout_passes=False),
        scratch_types=[
            pltpu.VMEM_SHARED((NPAD,), jnp.float32),
            pltpu.VMEM_SHARED((NPAD,), jnp.float32),
            pltpu.VMEM_SHARED((NPAD,), jnp.float32),
            pltpu.VMEM_SHARED((NPAD,), jnp.float32),
            pltpu.VMEM_SHARED((NPAD, D), jnp.float32),
            pltpu.VMEM((2, _CHUNK), jnp.int32),
            pltpu.VMEM((2, _CHUNK), jnp.int32),
            pltpu.VMEM((2, _CHUNK + _LANES), jnp.float32),
            pltpu.VMEM((2, _CHUNK), jnp.float32),
            pltpu.VMEM((2, _CHUNK), jnp.float32),
            pltpu.VMEM((2, _CHUNK), jnp.float32),
            pltpu.VMEM((2, _CHUNK), jnp.float32),
            pltpu.VMEM((_DCHUNK,), jnp.float32),
            pltpu.VMEM((DSUB, _IDXW), jnp.int32),
            pltpu.VMEM((RPT,), jnp.float32),
            pltpu.VMEM((2, _CHUNK, D), jnp.float32),
            pltpu.SemaphoreType.DMA,
            pltpu.SemaphoreType.DMA,
            pltpu.SemaphoreType.DMA,
            pltpu.SemaphoreType.DMA,
        ],
    )
    def sc_kernel(h_hbm, src_hbm, dst_hbm, a1_hbm, a2_hbm, mm_hbm,
                  deg_sh, a1_sh, a2_sh, d_sh, m_sh, src_v, dst_v, e_v,
                  a1g_v, a2g_v, ddg_v, dsg_v, ones_v, didx_v, z_v,
                  rows_v, sem0, sem1, ssem0, ssem1):
        sems = (sem0, sem1)
        ssems = (ssem0, ssem1)
        cid = lax.axis_index("c")
        sid = lax.axis_index("s")
        zeros = jnp.zeros((_LANES,), jnp.float32)
        ones = jnp.ones((_LANES,), jnp.float32)

        @pl.loop(0, RPT // _LANES)
        def _(i):
            z_v[pl.ds(i * _LANES, _LANES)] = zeros

        @pl.loop(0, _DCHUNK // _LANES)
        def _(i):
            ones_v[pl.ds(i * _LANES, _LANES)] = ones

        @pl.loop(0, _IDXW)
        def _(i):
            for k in range(D // _LANES):
                rows_v[0, i, pl.ds(k * _LANES, _LANES)] = zeros

        base = sid * RPT
        pltpu.sync_copy(z_v, deg_sh.at[pl.ds(base, RPT)])
        for r in range(RPT // _IDXW):
            pltpu.sync_copy(rows_v.at[0],
                            m_sh.at[pl.ds(base + r * _IDXW, _IDXW), :])
        pltpu.sync_copy(a1_hbm.at[pl.ds(base, RPT)], a1_sh.at[pl.ds(base, RPT)])
        pltpu.sync_copy(a2_hbm.at[pl.ds(base, RPT)], a2_sh.at[pl.ds(base, RPT)])
        plsc.subcore_barrier()

        drow0 = sid * T

        @pl.loop(0, nd16)
        def _(k):
            row = drow0 + k * DSUB
            pltpu.sync_copy(dst_hbm.at[pl.ds(row, DSUB), :], didx_v)
            for j in range(DSUB):
                pltpu.sync_copy(ones_v.at[pl.ds(j * _IDXW, _IDXW)],
                                deg_sh.at[didx_v.at[j]], add=True)
        if dtail:
            trow = drow0 + nd16 * DSUB
            pltpu.sync_copy(dst_hbm.at[pl.ds(trow, dtail), :],
                            didx_v.at[pl.ds(0, dtail), :])
            for j in range(dtail):
                pltpu.sync_copy(ones_v.at[pl.ds(j * _IDXW, _IDXW)],
                                deg_sh.at[didx_v.at[j]], add=True)
        plsc.subcore_barrier()

        pltpu.sync_copy(deg_sh.at[pl.ds(base, RPT)], z_v)

        @pl.loop(0, RPT // _LANES)
        def _(i):
            idx = lax.iota(jnp.int32, _LANES) + (base + i * _LANES)
            x = jnp.maximum(z_v[pl.ds(i * _LANES, _LANES)], 1.0)
            y = _rsqrt_newton(x)
            z_v[pl.ds(i * _LANES, _LANES)] = jnp.where(idx >= N, 0.0, y)

        pltpu.sync_copy(z_v, d_sh.at[pl.ds(base, RPT)])
        plsc.subcore_barrier()

        n_c = jnp.where(cid == 0, n0, n1)
        erow0 = jnp.where(cid == 0, sid * n0, _NS * n0 + sid * n1)

        def _prefetch(krow, nb):
            pltpu.sync_copy(src_hbm.at[krow], src_v.at[nb])
            pltpu.sync_copy(dst_hbm.at[krow], dst_v.at[nb])
            pltpu.async_copy(h_hbm.at[src_v.at[nb]], rows_v.at[nb], sems[nb])
            pltpu.async_copy(a1_sh.at[dst_v.at[nb]], a1g_v.at[nb], ssems[nb])
            pltpu.async_copy(a2_sh.at[src_v.at[nb]], a2g_v.at[nb], ssems[nb])
            pltpu.async_copy(d_sh.at[dst_v.at[nb]], ddg_v.at[nb], ssems[nb])
            pltpu.async_copy(d_sh.at[src_v.at[nb]], dsg_v.at[nb], ssems[nb])

        def _drain(b):
            pltpu.make_async_copy(a1_sh.at[dst_v.at[b]], a1g_v.at[b],
                                  ssems[b]).wait()
            pltpu.make_async_copy(a2_sh.at[src_v.at[b]], a2g_v.at[b],
                                  ssems[b]).wait()
            pltpu.make_async_copy(d_sh.at[dst_v.at[b]], ddg_v.at[b],
                                  ssems[b]).wait()
            pltpu.make_async_copy(d_sh.at[src_v.at[b]], dsg_v.at[b],
                                  ssems[b]).wait()

        _prefetch(erow0, 0)

        @pl.loop(0, n_c // 2)
        def _(p):
            for b in range(2):
                k = p * 2 + b
                nb = 1 - b
                nk = jnp.minimum(k + 1, n_c - 1)
                _prefetch(erow0 + nk, nb)
                _drain(b)
                for g in range(G):
                    sl = pl.ds(g * _LANES, _LANES)
                    e_v[b, sl] = (_tanh_via_exp(a1g_v[b, sl] + a2g_v[b, sl])
                                  * ddg_v[b, sl] * dsg_v[b, sl])
                pltpu.make_async_copy(h_hbm.at[src_v.at[b]],
                                      rows_v.at[b], sems[b]).wait()

                @plsc.parallel_loop(0, _CHUNK, unroll=4)
                def _(i):
                    es = e_v[b, pl.ds(i, _LANES)][0]
                    for kk in range(D // _LANES):
                        rows_v[b, i, pl.ds(kk * _LANES, _LANES)] = (
                            rows_v[b, i, pl.ds(kk * _LANES, _LANES)] * es)

                pltpu.sync_copy(rows_v.at[b], m_sh.at[dst_v.at[b]], add=True)

        pltpu.make_async_copy(h_hbm.at[src_v.at[0]], rows_v.at[0],
                              sems[0]).wait()
        _drain(0)
        plsc.subcore_barrier()

        for r in range(RPT // _IDXW):
            pltpu.sync_copy(m_sh.at[pl.ds(base + r * _IDXW, _IDXW), :],
                            mm_hbm.at[cid, pl.ds(base + r * _IDXW, _IDXW), :])

    return sc_kernel


def _pick_bs(n):
    for cand in (1024, 1000, 512, 500, 256, 250, 128, 125, 64, 40, 32, 25,
                 16, 10, 8, 5, 4, 2, 1):
        if n % cand == 0:
            return cand
    return 1


def _gate_proj(h, w2, b2):
    n, d = h.shape
    bs = _pick_bs(n)

    def body(h_ref, w_ref, b_ref, o_ref):
        o_ref[...] = jnp.dot(h_ref[...], w_ref[...],
                             preferred_element_type=jnp.float32) + b_ref[...]

    return pl.pallas_call(
        body,
        grid=(n // bs,),
        in_specs=[pl.BlockSpec((bs, d), lambda i: (i, 0)),
                  pl.BlockSpec((d, 2), lambda i: (0, 0)),
                  pl.BlockSpec((1, 2), lambda i: (0, 0))],
        out_specs=pl.BlockSpec((bs, 2), lambda i: (i, 0)),
        out_shape=jax.ShapeDtypeStruct((n, 2), jnp.float32),
    )(h, w2, b2)


def _combine(h, mm):
    n, d = h.shape
    bs = _pick_bs(n)

    def body(h_ref, m0_ref, m1_ref, o_ref):
        o_ref[...] = jnp.maximum(
            _EPS * h_ref[...] + m0_ref[0] + m1_ref[0], 0.0)

    return pl.pallas_call(
        body,
        grid=(n // bs,),
        in_specs=[pl.BlockSpec((bs, d), lambda i: (i, 0)),
                  pl.BlockSpec((1, bs, d), lambda i: (0, i, 0)),
                  pl.BlockSpec((1, bs, d), lambda i: (1, i, 0))],
        out_specs=pl.BlockSpec((bs, d), lambda i: (i, 0)),
        out_shape=jax.ShapeDtypeStruct((n, d), jnp.float32),
    )(h, mm, mm)


_SC0_FRAC = 0.71


def kernel(h, edge_index, gate_w, gate_b):
    n, d = h.shape
    e = edge_index.shape[1]

    npad = -((n + 1) // -(_NS * _LANES)) * (_NS * _LANES)
    per_chunk = _NS * _CHUNK
    t = -(e // -(per_chunk * 16)) * 16
    n0 = int(round(t * _SC0_FRAC / 2)) * 2
    n0 = min(max(n0, 2), t - 2)
    n1 = t - n0
    e_pad = t * per_chunk

    src = edge_index[0]
    dst = edge_index[1]
    pad = e_pad - e
    srcp = jnp.concatenate(
        [src, jnp.zeros((pad,), jnp.int32)]).reshape(e_pad // _IDXW, _IDXW)
    dstp = jnp.concatenate(
        [dst, jnp.full((pad,), n, jnp.int32)]).reshape(e_pad // _IDXW, _IDXW)

    w_dst = gate_w[:d, 0]
    w_src = gate_w[d:, 0]
    w2 = jnp.stack([w_dst, w_src], axis=1)
    b2 = jnp.stack([gate_b[0], jnp.zeros((), jnp.float32)]).reshape(1, 2)

    a = _gate_proj(h, w2, b2)
    a1 = jnp.pad(a[:, 0], (0, npad - n))
    a2 = jnp.pad(a[:, 1], (0, npad - n))

    mm = _make_sc_kernel(n, d, npad, n0, n1)(h, srcp, dstp, a1, a2)
    return _combine(h, mm)


# speedup vs baseline: 1.7227x; 1.3589x over previous
import functools

import jax
import jax.numpy as jnp
from jax import lax
from jax.experimental import pallas as pl
from jax.experimental.pallas import tpu as pltpu
from jax.experimental.pallas import tpu_sc as plsc

_EPS = 0.3
_NC = 2
_NS = 16
_LANES = 16
_CHUNK = 128
_DCHUNK = 2048
_IDXW = 128


def _rsqrt_newton(x):
    xi = lax.bitcast_convert_type(x, jnp.int32)
    yi = jnp.int32(0x5F3759DF) - (xi >> 1)
    y = lax.bitcast_convert_type(yi, jnp.float32)
    for _ in range(3):
        y = y * (1.5 - 0.5 * x * y * y)
    return y


def _tanh_via_exp(x):
    t = jnp.exp(-2.0 * jnp.abs(x))
    th = (1.0 - t) / (1.0 + t)
    return jnp.where(x < 0.0, -th, th)


def _make_sc_kernel(N, D, NPAD, n0, n1):
    RPT = NPAD // _NS
    T = n0 + n1
    DSUB = _DCHUNK // _IDXW
    nd16 = T // DSUB
    dtail = T % DSUB
    G = _CHUNK // _LANES

    mesh = plsc.VectorSubcoreMesh(
        core_axis_name="c", subcore_axis_name="s",
        num_cores=_NC, num_subcores=_NS)

    @functools.partial(
        pl.kernel,
        out_type=jax.ShapeDtypeStruct((_NC, NPAD, D), jnp.float32),
        mesh=mesh,
        compiler_params=pltpu.CompilerParams(needs_layout_passes=False,
                                             use_tc_tiling_on_sc=False),
        scratch_types=[
            pltpu.VMEM_SHARED((NPAD,), jnp.float32),
            pltpu.VMEM_SHARED((NPAD,), jnp.float32),
            pltpu.VMEM_SHARED((NPAD,), jnp.float32),
            pltpu.VMEM_SHARED((NPAD,), jnp.float32),
            pltpu.VMEM_SHARED((NPAD, D), jnp.float32),
            pltpu.VMEM((2, _CHUNK), jnp.int32),
            pltpu.VMEM((2, _CHUNK), jnp.int32),
            pltpu.VMEM((2, _CHUNK + _LANES), jnp.float32),
            pltpu.VMEM((2, _CHUNK), jnp.float32),
            pltpu.VMEM((2, _CHUNK), jnp.float32),
            pltpu.VMEM((2, _CHUNK), jnp.float32),
            pltpu.VMEM((2, _CHUNK), jnp.float32),
            pltpu.VMEM((_DCHUNK,), jnp.float32),
            pltpu.VMEM((DSUB, _IDXW), jnp.int32),
            pltpu.VMEM((RPT,), jnp.float32),
            pltpu.VMEM((2, _CHUNK, D // 2), jnp.int32),
            pltpu.VMEM((_CHUNK, D), jnp.float32),
            pltpu.SemaphoreType.DMA,
            pltpu.SemaphoreType.DMA,
            pltpu.SemaphoreType.DMA,
            pltpu.SemaphoreType.DMA,
        ],
    )
    def sc_kernel(h_hbm, src_hbm, dst_hbm, a1_hbm, a2_hbm, mm_hbm,
                  deg_sh, a1_sh, a2_sh, d_sh, m_sh, src_v, dst_v, e_v,
                  a1g_v, a2g_v, ddg_v, dsg_v, ones_v, didx_v, z_v,
                  rows_v, frows_v, sem0, sem1, ssem0, ssem1):
        sems = (sem0, sem1)
        ssems = (ssem0, ssem1)
        cid = lax.axis_index("c")
        sid = lax.axis_index("s")
        zeros = jnp.zeros((_LANES,), jnp.float32)
        ones = jnp.ones((_LANES,), jnp.float32)

        @pl.loop(0, RPT // _LANES)
        def _(i):
            z_v[pl.ds(i * _LANES, _LANES)] = zeros

        @pl.loop(0, _DCHUNK // _LANES)
        def _(i):
            ones_v[pl.ds(i * _LANES, _LANES)] = ones

        @pl.loop(0, _IDXW)
        def _(i):
            for k in range(D // _LANES):
                frows_v[i, pl.ds(k * _LANES, _LANES)] = zeros

        base = sid * RPT
        pltpu.sync_copy(z_v, deg_sh.at[pl.ds(base, RPT)])
        for r in range(RPT // _IDXW):
            pltpu.sync_copy(frows_v.at[pl.ds(0, _IDXW), :],
                            m_sh.at[pl.ds(base + r * _IDXW, _IDXW), :])
        pltpu.sync_copy(a1_hbm.at[pl.ds(base, RPT)], a1_sh.at[pl.ds(base, RPT)])
        pltpu.sync_copy(a2_hbm.at[pl.ds(base, RPT)], a2_sh.at[pl.ds(base, RPT)])
        plsc.subcore_barrier()

        drow0 = sid * T

        @pl.loop(0, nd16)
        def _(k):
            row = drow0 + k * DSUB
            pltpu.sync_copy(dst_hbm.at[pl.ds(row, DSUB), :], didx_v)
            for j in range(DSUB):
                pltpu.sync_copy(ones_v.at[pl.ds(j * _IDXW, _IDXW)],
                                deg_sh.at[didx_v.at[j]], add=True)
        if dtail:
            trow = drow0 + nd16 * DSUB
            pltpu.sync_copy(dst_hbm.at[pl.ds(trow, dtail), :],
                            didx_v.at[pl.ds(0, dtail), :])
            for j in range(dtail):
                pltpu.sync_copy(ones_v.at[pl.ds(j * _IDXW, _IDXW)],
                                deg_sh.at[didx_v.at[j]], add=True)
        plsc.subcore_barrier()

        pltpu.sync_copy(deg_sh.at[pl.ds(base, RPT)], z_v)

        @pl.loop(0, RPT // _LANES)
        def _(i):
            idx = lax.iota(jnp.int32, _LANES) + (base + i * _LANES)
            x = jnp.maximum(z_v[pl.ds(i * _LANES, _LANES)], 1.0)
            y = _rsqrt_newton(x)
            z_v[pl.ds(i * _LANES, _LANES)] = jnp.where(idx >= N, 0.0, y)

        pltpu.sync_copy(z_v, d_sh.at[pl.ds(base, RPT)])
        plsc.subcore_barrier()

        n_c = jnp.where(cid == 0, n0, n1)
        erow0 = jnp.where(cid == 0, sid * n0, _NS * n0 + sid * n1)

        def _prefetch(krow, nb):
            pltpu.sync_copy(src_hbm.at[krow], src_v.at[nb])
            pltpu.sync_copy(dst_hbm.at[krow], dst_v.at[nb])
            pltpu.async_copy(h_hbm.at[src_v.at[nb]], rows_v.at[nb], sems[nb])
            pltpu.async_copy(a1_sh.at[dst_v.at[nb]], a1g_v.at[nb], ssems[nb])
            pltpu.async_copy(a2_sh.at[src_v.at[nb]], a2g_v.at[nb], ssems[nb])
            pltpu.async_copy(d_sh.at[dst_v.at[nb]], ddg_v.at[nb], ssems[nb])
            pltpu.async_copy(d_sh.at[src_v.at[nb]], dsg_v.at[nb], ssems[nb])

        def _drain(b):
            pltpu.make_async_copy(a1_sh.at[dst_v.at[b]], a1g_v.at[b],
                                  ssems[b]).wait()
            pltpu.make_async_copy(a2_sh.at[src_v.at[b]], a2g_v.at[b],
                                  ssems[b]).wait()
            pltpu.make_async_copy(d_sh.at[dst_v.at[b]], ddg_v.at[b],
                                  ssems[b]).wait()
            pltpu.make_async_copy(d_sh.at[src_v.at[b]], dsg_v.at[b],
                                  ssems[b]).wait()

        def _drain_rows(b):
            pltpu.make_async_copy(h_hbm.at[src_v.at[b]],
                                  rows_v.at[b], sems[b]).wait()

        _prefetch(erow0, 0)

        @pl.loop(0, n_c // 2)
        def _(p):
            for b in range(2):
                k = p * 2 + b
                nb = 1 - b
                nk = jnp.minimum(k + 1, n_c - 1)
                _prefetch(erow0 + nk, nb)
                _drain(b)
                for g in range(G):
                    sl = pl.ds(g * _LANES, _LANES)
                    e_v[b, sl] = (_tanh_via_exp(a1g_v[b, sl] + a2g_v[b, sl])
                                  * ddg_v[b, sl] * dsg_v[b, sl])
                _drain_rows(b)

                @plsc.parallel_loop(0, _CHUNK, unroll=4)
                def _(i):
                    es = e_v[b, pl.ds(i, _LANES)][0]
                    for kk in range(D // (2 * _LANES)):
                        x32 = rows_v[b, i, pl.ds(kk * _LANES, _LANES)]
                        x = plsc.bitcast(x32, jnp.bfloat16)
                        pa, pb = plsc.unpack(
                            x, format=plsc.PackFormat.INTERLEAVED)
                        frows_v[i, pl.ds(kk * 2 * _LANES, _LANES)] = pa * es
                        frows_v[i, pl.ds(kk * 2 * _LANES + _LANES,
                                         _LANES)] = pb * es

                pltpu.sync_copy(frows_v, m_sh.at[dst_v.at[b]], add=True)

        _drain(0)
        _drain_rows(0)
        plsc.subcore_barrier()

        for r in range(RPT // _IDXW):
            pltpu.sync_copy(m_sh.at[pl.ds(base + r * _IDXW, _IDXW), :],
                            mm_hbm.at[cid, pl.ds(base + r * _IDXW, _IDXW), :])

    return sc_kernel


def _pick_bs(n):
    for cand in (1024, 1000, 512, 500, 256, 250, 128, 125, 64, 40, 32, 25,
                 16, 10, 8, 5, 4, 2, 1):
        if n % cand == 0:
            return cand
    return 1


def _gate_proj(h, w2, b2):
    n, d = h.shape
    bs = _pick_bs(n)

    def body(h_ref, w_ref, b_ref, o_ref):
        o_ref[...] = jnp.dot(h_ref[...], w_ref[...],
                             preferred_element_type=jnp.float32) + b_ref[...]

    return pl.pallas_call(
        body,
        grid=(n // bs,),
        in_specs=[pl.BlockSpec((bs, d), lambda i: (i, 0)),
                  pl.BlockSpec((d, 2), lambda i: (0, 0)),
                  pl.BlockSpec((1, 2), lambda i: (0, 0))],
        out_specs=pl.BlockSpec((bs, 2), lambda i: (i, 0)),
        out_shape=jax.ShapeDtypeStruct((n, 2), jnp.float32),
    )(h, w2, b2)


def _combine(h, mm):
    n, d = h.shape
    bs = _pick_bs(n)

    def body(h_ref, m0_ref, m1_ref, o_ref):
        o_ref[...] = jnp.maximum(
            _EPS * h_ref[...] + m0_ref[0] + m1_ref[0], 0.0)

    return pl.pallas_call(
        body,
        grid=(n // bs,),
        in_specs=[pl.BlockSpec((bs, d), lambda i: (i, 0)),
                  pl.BlockSpec((1, bs, d), lambda i: (0, i, 0)),
                  pl.BlockSpec((1, bs, d), lambda i: (1, i, 0))],
        out_specs=pl.BlockSpec((bs, d), lambda i: (i, 0)),
        out_shape=jax.ShapeDtypeStruct((n, d), jnp.float32),
    )(h, mm, mm)


_SC0_FRAC = 0.71


def kernel(h, edge_index, gate_w, gate_b):
    n, d = h.shape
    e = edge_index.shape[1]

    npad = -((n + 1) // -(_NS * _LANES)) * (_NS * _LANES)
    per_chunk = _NS * _CHUNK
    t = -(e // -(per_chunk * 16)) * 16
    n0 = int(round(t * _SC0_FRAC / 2)) * 2
    n0 = min(max(n0, 2), t - 2)
    n1 = t - n0
    e_pad = t * per_chunk

    src = edge_index[0]
    dst = edge_index[1]
    pad = e_pad - e
    srcp = jnp.concatenate(
        [src, jnp.zeros((pad,), jnp.int32)]).reshape(e_pad // _IDXW, _IDXW)
    dstp = jnp.concatenate(
        [dst, jnp.full((pad,), n, jnp.int32)]).reshape(e_pad // _IDXW, _IDXW)

    w_dst = gate_w[:d, 0]
    w_src = gate_w[d:, 0]
    w2 = jnp.stack([w_dst, w_src], axis=1)
    b2 = jnp.stack([gate_b[0], jnp.zeros((), jnp.float32)]).reshape(1, 2)

    a = _gate_proj(h, w2, b2)
    a1 = jnp.pad(a[:, 0], (0, npad - n))
    a2 = jnp.pad(a[:, 1], (0, npad - n))

    hbf = (h.reshape(n, d // 32, 2, 16).transpose(0, 1, 3, 2).reshape(n, d)
           .astype(jnp.bfloat16))
    hbf = jnp.pad(hbf, ((0, npad - n), (0, 0)))
    hbf = lax.bitcast_convert_type(hbf.reshape(npad, d // 2, 2), jnp.int32)

    mm = _make_sc_kernel(n, d, npad, n0, n1)(hbf, srcp, dstp, a1, a2)
    return _combine(h, mm)


# speedup vs baseline: 1.8481x; 1.0728x over previous
import functools

import jax
import jax.numpy as jnp
from jax import lax
from jax.experimental import pallas as pl
from jax.experimental.pallas import tpu as pltpu
from jax.experimental.pallas import tpu_sc as plsc

_EPS = 0.3
_NC = 2
_NS = 16
_LANES = 16
_CHUNK = 128
_DCHUNK = 2048
_IDXW = 128


def _rsqrt_newton(x):
    xi = lax.bitcast_convert_type(x, jnp.int32)
    yi = jnp.int32(0x5F3759DF) - (xi >> 1)
    y = lax.bitcast_convert_type(yi, jnp.float32)
    for _ in range(3):
        y = y * (1.5 - 0.5 * x * y * y)
    return y


def _tanh_via_exp(x):
    t = jnp.exp(-2.0 * jnp.abs(x))
    th = (1.0 - t) / (1.0 + t)
    return jnp.where(x < 0.0, -th, th)


def _make_sc_kernel(N, D, NPAD, n0, n1):
    RPT = NPAD // _NS
    T = n0 + n1
    DSUB = _DCHUNK // _IDXW
    nd16 = T // DSUB
    dtail = T % DSUB
    G = _CHUNK // _LANES

    mesh = plsc.VectorSubcoreMesh(
        core_axis_name="c", subcore_axis_name="s",
        num_cores=_NC, num_subcores=_NS)

    @functools.partial(
        pl.kernel,
        out_type=jax.ShapeDtypeStruct((_NC, NPAD, D), jnp.float32),
        mesh=mesh,
        compiler_params=pltpu.CompilerParams(needs_layout_passes=False,
                                             use_tc_tiling_on_sc=False),
        scratch_types=[
            pltpu.VMEM_SHARED((NPAD,), jnp.float32),
            pltpu.VMEM_SHARED((NPAD,), jnp.float32),
            pltpu.VMEM_SHARED((NPAD,), jnp.float32),
            pltpu.VMEM_SHARED((NPAD,), jnp.float32),
            pltpu.VMEM_SHARED((NPAD, D), jnp.float32),
            pltpu.VMEM((2, _CHUNK), jnp.int32),
            pltpu.VMEM((2, _CHUNK), jnp.int32),
            pltpu.VMEM((2, _CHUNK + _LANES), jnp.float32),
            pltpu.VMEM((2, _CHUNK), jnp.float32),
            pltpu.VMEM((2, _CHUNK), jnp.float32),
            pltpu.VMEM((2, _CHUNK), jnp.float32),
            pltpu.VMEM((2, _CHUNK), jnp.float32),
            pltpu.VMEM((_DCHUNK,), jnp.float32),
            pltpu.VMEM((DSUB, _IDXW), jnp.int32),
            pltpu.VMEM((RPT,), jnp.float32),
            pltpu.VMEM((2, _CHUNK, D // 2), jnp.int32),
            pltpu.VMEM((_CHUNK, D), jnp.float32),
            pltpu.SemaphoreType.DMA,
            pltpu.SemaphoreType.DMA,
            pltpu.SemaphoreType.DMA,
            pltpu.SemaphoreType.DMA,
        ],
    )
    def sc_kernel(h_hbm, src_hbm, dst_hbm, a1_hbm, a2_hbm, mm_hbm,
                  deg_sh, a1_sh, a2_sh, d_sh, m_sh, src_v, dst_v, e_v,
                  a1g_v, a2g_v, ddg_v, dsg_v, ones_v, didx_v, z_v,
                  rows_v, frows_v, sem0, sem1, ssem0, ssem1):
        sems = (sem0, sem1)
        ssems = (ssem0, ssem1)
        cid = lax.axis_index("c")
        sid = lax.axis_index("s")
        zeros = jnp.zeros((_LANES,), jnp.float32)
        ones = jnp.ones((_LANES,), jnp.float32)

        @pl.loop(0, RPT // _LANES)
        def _(i):
            z_v[pl.ds(i * _LANES, _LANES)] = zeros

        @pl.loop(0, _DCHUNK // _LANES)
        def _(i):
            ones_v[pl.ds(i * _LANES, _LANES)] = ones

        @pl.loop(0, _IDXW)
        def _(i):
            for k in range(D // _LANES):
                frows_v[i, pl.ds(k * _LANES, _LANES)] = zeros

        base = sid * RPT
        pltpu.sync_copy(z_v, deg_sh.at[pl.ds(base, RPT)])
        for r in range(RPT // _IDXW):
            pltpu.sync_copy(frows_v.at[pl.ds(0, _IDXW), :],
                            m_sh.at[pl.ds(base + r * _IDXW, _IDXW), :])
        pltpu.sync_copy(a1_hbm.at[pl.ds(base, RPT)], a1_sh.at[pl.ds(base, RPT)])
        pltpu.sync_copy(a2_hbm.at[pl.ds(base, RPT)], a2_sh.at[pl.ds(base, RPT)])
        plsc.subcore_barrier()

        drow0 = sid * T

        @pl.loop(0, nd16)
        def _(k):
            row = drow0 + k * DSUB
            pltpu.sync_copy(dst_hbm.at[pl.ds(row, DSUB), :], didx_v)
            for j in range(DSUB):
                pltpu.sync_copy(ones_v.at[pl.ds(j * _IDXW, _IDXW)],
                                deg_sh.at[didx_v.at[j]], add=True)
        if dtail:
            trow = drow0 + nd16 * DSUB
            pltpu.sync_copy(dst_hbm.at[pl.ds(trow, dtail), :],
                            didx_v.at[pl.ds(0, dtail), :])
            for j in range(dtail):
                pltpu.sync_copy(ones_v.at[pl.ds(j * _IDXW, _IDXW)],
                                deg_sh.at[didx_v.at[j]], add=True)
        plsc.subcore_barrier()

        pltpu.sync_copy(deg_sh.at[pl.ds(base, RPT)], z_v)

        @pl.loop(0, RPT // _LANES)
        def _(i):
            idx = lax.iota(jnp.int32, _LANES) + (base + i * _LANES)
            x = jnp.maximum(z_v[pl.ds(i * _LANES, _LANES)], 1.0)
            y = _rsqrt_newton(x)
            z_v[pl.ds(i * _LANES, _LANES)] = jnp.where(idx >= N, 0.0, y)

        pltpu.sync_copy(z_v, d_sh.at[pl.ds(base, RPT)])
        plsc.subcore_barrier()

        n_c = jnp.where(cid == 0, n0, n1)
        erow0 = jnp.where(cid == 0, sid * n0, _NS * n0 + sid * n1)

        def _prefetch(krow, nb):
            pltpu.sync_copy(src_hbm.at[krow], src_v.at[nb])
            pltpu.sync_copy(dst_hbm.at[krow], dst_v.at[nb])
            pltpu.async_copy(h_hbm.at[src_v.at[nb]], rows_v.at[nb], sems[nb])
            pltpu.async_copy(a1_sh.at[dst_v.at[nb]], a1g_v.at[nb], ssems[nb])
            pltpu.async_copy(a2_sh.at[src_v.at[nb]], a2g_v.at[nb], ssems[nb])
            pltpu.async_copy(d_sh.at[dst_v.at[nb]], ddg_v.at[nb], ssems[nb])
            pltpu.async_copy(d_sh.at[src_v.at[nb]], dsg_v.at[nb], ssems[nb])

        def _drain(b):
            pltpu.make_async_copy(a1_sh.at[dst_v.at[b]], a1g_v.at[b],
                                  ssems[b]).wait()
            pltpu.make_async_copy(a2_sh.at[src_v.at[b]], a2g_v.at[b],
                                  ssems[b]).wait()
            pltpu.make_async_copy(d_sh.at[dst_v.at[b]], ddg_v.at[b],
                                  ssems[b]).wait()
            pltpu.make_async_copy(d_sh.at[src_v.at[b]], dsg_v.at[b],
                                  ssems[b]).wait()

        def _drain_rows(b):
            pltpu.make_async_copy(h_hbm.at[src_v.at[b]],
                                  rows_v.at[b], sems[b]).wait()

        _prefetch(erow0, 0)

        @pl.loop(0, n_c // 2)
        def _(p):
            for b in range(2):
                k = p * 2 + b
                nb = 1 - b
                nk = jnp.minimum(k + 1, n_c - 1)
                _prefetch(erow0 + nk, nb)
                _drain(b)
                for g in range(G):
                    sl = pl.ds(g * _LANES, _LANES)
                    e_v[b, sl] = (_tanh_via_exp(a1g_v[b, sl] + a2g_v[b, sl])
                                  * ddg_v[b, sl] * dsg_v[b, sl])
                _drain_rows(b)

                @plsc.parallel_loop(0, _CHUNK, unroll=4)
                def _(i):
                    es = e_v[b, pl.ds(i, _LANES)][0]
                    for kk in range(D // (2 * _LANES)):
                        x32 = rows_v[b, i, pl.ds(kk * _LANES, _LANES)]
                        x = plsc.bitcast(x32, jnp.bfloat16)
                        pa, pb = plsc.unpack(
                            x, format=plsc.PackFormat.INTERLEAVED)
                        frows_v[i, pl.ds(kk * 2 * _LANES, _LANES)] = pa * es
                        frows_v[i, pl.ds(kk * 2 * _LANES + _LANES,
                                         _LANES)] = pb * es

                pltpu.sync_copy(frows_v, m_sh.at[dst_v.at[b]], add=True)

        _drain(0)
        _drain_rows(0)
        plsc.subcore_barrier()

        for r in range(RPT // _IDXW):
            pltpu.sync_copy(m_sh.at[pl.ds(base + r * _IDXW, _IDXW), :],
                            mm_hbm.at[cid, pl.ds(base + r * _IDXW, _IDXW), :])

    return sc_kernel


def _pick_bs(n):
    for cand in (1024, 1000, 512, 500, 256, 250, 128, 125, 64, 40, 32, 25,
                 16, 10, 8, 5, 4, 2, 1):
        if n % cand == 0:
            return cand
    return 1


def _gate_proj(h, w2, b2):
    n, d = h.shape
    bs = _pick_bs(n)

    def body(h_ref, w_ref, b_ref, o_ref):
        o_ref[...] = jnp.dot(h_ref[...], w_ref[...],
                             preferred_element_type=jnp.float32) + b_ref[...]

    return pl.pallas_call(
        body,
        grid=(n // bs,),
        in_specs=[pl.BlockSpec((bs, d), lambda i: (i, 0)),
                  pl.BlockSpec((d, 2), lambda i: (0, 0)),
                  pl.BlockSpec((1, 2), lambda i: (0, 0))],
        out_specs=pl.BlockSpec((bs, 2), lambda i: (i, 0)),
        out_shape=jax.ShapeDtypeStruct((n, 2), jnp.float32),
    )(h, w2, b2)


def _combine(h, mm):
    n, d = h.shape
    bs = _pick_bs(n)

    def body(h_ref, m0_ref, m1_ref, o_ref):
        o_ref[...] = jnp.maximum(
            _EPS * h_ref[...] + m0_ref[0] + m1_ref[0], 0.0)

    return pl.pallas_call(
        body,
        grid=(n // bs,),
        in_specs=[pl.BlockSpec((bs, d), lambda i: (i, 0)),
                  pl.BlockSpec((1, bs, d), lambda i: (0, i, 0)),
                  pl.BlockSpec((1, bs, d), lambda i: (1, i, 0))],
        out_specs=pl.BlockSpec((bs, d), lambda i: (i, 0)),
        out_shape=jax.ShapeDtypeStruct((n, d), jnp.float32),
    )(h, mm, mm)


_SC0_FRAC = 0.64


def kernel(h, edge_index, gate_w, gate_b):
    n, d = h.shape
    e = edge_index.shape[1]

    npad = -((n + 1) // -(_NS * _LANES)) * (_NS * _LANES)
    per_chunk = _NS * _CHUNK
    t = -(e // -(per_chunk * 16)) * 16
    n0 = int(round(t * _SC0_FRAC / 2)) * 2
    n0 = min(max(n0, 2), t - 2)
    n1 = t - n0
    e_pad = t * per_chunk

    src = edge_index[0]
    dst = edge_index[1]
    pad = e_pad - e
    srcp = jnp.concatenate(
        [src, jnp.zeros((pad,), jnp.int32)]).reshape(e_pad // _IDXW, _IDXW)
    dstp = jnp.concatenate(
        [dst, jnp.full((pad,), n, jnp.int32)]).reshape(e_pad // _IDXW, _IDXW)

    w_dst = gate_w[:d, 0]
    w_src = gate_w[d:, 0]
    w2 = jnp.stack([w_dst, w_src], axis=1)
    b2 = jnp.stack([gate_b[0], jnp.zeros((), jnp.float32)]).reshape(1, 2)

    a = _gate_proj(h, w2, b2)
    a1 = jnp.pad(a[:, 0], (0, npad - n))
    a2 = jnp.pad(a[:, 1], (0, npad - n))

    hbf = (h.reshape(n, d // 32, 2, 16).transpose(0, 1, 3, 2).reshape(n, d)
           .astype(jnp.bfloat16))
    hbf = jnp.pad(hbf, ((0, npad - n), (0, 0)))
    hbf = lax.bitcast_convert_type(hbf.reshape(npad, d // 2, 2), jnp.int32)

    mm = _make_sc_kernel(n, d, npad, n0, n1)(hbf, srcp, dstp, a1, a2)
    return _combine(h, mm)


# speedup vs baseline: 1.9188x; 1.0383x over previous
import functools

import jax
import jax.numpy as jnp
from jax import lax
from jax.experimental import pallas as pl
from jax.experimental.pallas import tpu as pltpu
from jax.experimental.pallas import tpu_sc as plsc

_EPS = 0.3
_NC = 2
_NS = 16
_LANES = 16
_CHUNK = 128
_DCHUNK = 2048
_IDXW = 128


def _rsqrt_newton(x):
    xi = lax.bitcast_convert_type(x, jnp.int32)
    yi = jnp.int32(0x5F3759DF) - (xi >> 1)
    y = lax.bitcast_convert_type(yi, jnp.float32)
    for _ in range(3):
        y = y * (1.5 - 0.5 * x * y * y)
    return y


def _tanh_via_exp(x):
    t = jnp.exp(-2.0 * jnp.abs(x))
    th = (1.0 - t) / (1.0 + t)
    return jnp.where(x < 0.0, -th, th)


def _make_sc_kernel(N, D, NPAD, n0, n1):
    RPT = NPAD // _NS
    T = n0 + n1
    DSUB = _DCHUNK // _IDXW
    nd16 = T // DSUB
    dtail = T % DSUB
    G = _CHUNK // _LANES

    mesh = plsc.VectorSubcoreMesh(
        core_axis_name="c", subcore_axis_name="s",
        num_cores=_NC, num_subcores=_NS)

    @functools.partial(
        pl.kernel,
        out_type=jax.ShapeDtypeStruct((_NC, NPAD, D), jnp.float32),
        mesh=mesh,
        compiler_params=pltpu.CompilerParams(needs_layout_passes=False,
                                             use_tc_tiling_on_sc=False),
        scratch_types=[
            pltpu.VMEM_SHARED((NPAD,), jnp.float32),
            pltpu.VMEM_SHARED((NPAD,), jnp.float32),
            pltpu.VMEM_SHARED((NPAD,), jnp.float32),
            pltpu.VMEM_SHARED((NPAD,), jnp.float32),
            pltpu.VMEM_SHARED((NPAD, D), jnp.float32),
            pltpu.VMEM((2, _CHUNK), jnp.int32),
            pltpu.VMEM((2, _CHUNK), jnp.int32),
            pltpu.VMEM((2, _CHUNK + _LANES), jnp.float32),
            pltpu.VMEM((2, _CHUNK), jnp.float32),
            pltpu.VMEM((2, _CHUNK), jnp.float32),
            pltpu.VMEM((2, _CHUNK), jnp.float32),
            pltpu.VMEM((2, _CHUNK), jnp.float32),
            pltpu.VMEM((_DCHUNK,), jnp.float32),
            pltpu.VMEM((DSUB, _IDXW), jnp.int32),
            pltpu.VMEM((RPT,), jnp.float32),
            pltpu.VMEM((2, _CHUNK, D // 2), jnp.int32),
            pltpu.VMEM((_CHUNK, D), jnp.float32),
            pltpu.SemaphoreType.DMA,
            pltpu.SemaphoreType.DMA,
            pltpu.SemaphoreType.DMA,
            pltpu.SemaphoreType.DMA,
        ],
    )
    def sc_kernel(h_hbm, src_hbm, dst_hbm, a1_hbm, a2_hbm, mm_hbm,
                  deg_sh, a1_sh, a2_sh, d_sh, m_sh, src_v, dst_v, e_v,
                  a1g_v, a2g_v, ddg_v, dsg_v, ones_v, didx_v, z_v,
                  rows_v, frows_v, sem0, sem1, ssem0, ssem1):
        sems = (sem0, sem1)
        ssems = (ssem0, ssem1)
        cid = lax.axis_index("c")
        sid = lax.axis_index("s")
        zeros = jnp.zeros((_LANES,), jnp.float32)
        ones = jnp.ones((_LANES,), jnp.float32)

        @pl.loop(0, RPT // _LANES)
        def _(i):
            z_v[pl.ds(i * _LANES, _LANES)] = zeros

        @pl.loop(0, _DCHUNK // _LANES)
        def _(i):
            ones_v[pl.ds(i * _LANES, _LANES)] = ones

        @pl.loop(0, _IDXW)
        def _(i):
            for k in range(D // _LANES):
                frows_v[i, pl.ds(k * _LANES, _LANES)] = zeros

        base = sid * RPT
        pltpu.sync_copy(z_v, deg_sh.at[pl.ds(base, RPT)])
        for r in range(RPT // _IDXW):
            pltpu.sync_copy(frows_v.at[pl.ds(0, _IDXW), :],
                            m_sh.at[pl.ds(base + r * _IDXW, _IDXW), :])
        pltpu.sync_copy(a1_hbm.at[pl.ds(base, RPT)], a1_sh.at[pl.ds(base, RPT)])
        pltpu.sync_copy(a2_hbm.at[pl.ds(base, RPT)], a2_sh.at[pl.ds(base, RPT)])
        plsc.subcore_barrier()

        drow0 = sid * T

        @pl.loop(0, nd16)
        def _(k):
            row = drow0 + k * DSUB
            pltpu.sync_copy(dst_hbm.at[pl.ds(row, DSUB), :], didx_v)
            for j in range(DSUB):
                pltpu.sync_copy(ones_v.at[pl.ds(j * _IDXW, _IDXW)],
                                deg_sh.at[didx_v.at[j]], add=True)
        if dtail:
            trow = drow0 + nd16 * DSUB
            pltpu.sync_copy(dst_hbm.at[pl.ds(trow, dtail), :],
                            didx_v.at[pl.ds(0, dtail), :])
            for j in range(dtail):
                pltpu.sync_copy(ones_v.at[pl.ds(j * _IDXW, _IDXW)],
                                deg_sh.at[didx_v.at[j]], add=True)
        plsc.subcore_barrier()

        pltpu.sync_copy(deg_sh.at[pl.ds(base, RPT)], z_v)

        @pl.loop(0, RPT // _LANES)
        def _(i):
            idx = lax.iota(jnp.int32, _LANES) + (base + i * _LANES)
            x = jnp.maximum(z_v[pl.ds(i * _LANES, _LANES)], 1.0)
            y = _rsqrt_newton(x)
            z_v[pl.ds(i * _LANES, _LANES)] = jnp.where(idx >= N, 0.0, y)

        pltpu.sync_copy(z_v, d_sh.at[pl.ds(base, RPT)])
        plsc.subcore_barrier()

        n_c = jnp.where(cid == 0, n0, n1)
        erow0 = jnp.where(cid == 0, sid * n0, _NS * n0 + sid * n1)

        def _prefetch(krow, nb):
            pltpu.sync_copy(src_hbm.at[krow], src_v.at[nb])
            pltpu.sync_copy(dst_hbm.at[krow], dst_v.at[nb])
            pltpu.async_copy(h_hbm.at[src_v.at[nb]], rows_v.at[nb], sems[nb])
            pltpu.async_copy(a1_sh.at[dst_v.at[nb]], a1g_v.at[nb], ssems[nb])
            pltpu.async_copy(a2_sh.at[src_v.at[nb]], a2g_v.at[nb], ssems[nb])
            pltpu.async_copy(d_sh.at[dst_v.at[nb]], ddg_v.at[nb], ssems[nb])
            pltpu.async_copy(d_sh.at[src_v.at[nb]], dsg_v.at[nb], ssems[nb])

        def _drain(b):
            pltpu.make_async_copy(a1_sh.at[dst_v.at[b]], a1g_v.at[b],
                                  ssems[b]).wait()
            pltpu.make_async_copy(a2_sh.at[src_v.at[b]], a2g_v.at[b],
                                  ssems[b]).wait()
            pltpu.make_async_copy(d_sh.at[dst_v.at[b]], ddg_v.at[b],
                                  ssems[b]).wait()
            pltpu.make_async_copy(d_sh.at[src_v.at[b]], dsg_v.at[b],
                                  ssems[b]).wait()

        def _drain_rows(b):
            pltpu.make_async_copy(h_hbm.at[src_v.at[b]],
                                  rows_v.at[b], sems[b]).wait()

        _prefetch(erow0, 0)

        @pl.loop(0, n_c // 2)
        def _(p):
            for b in range(2):
                k = p * 2 + b
                nb = 1 - b
                nk = jnp.minimum(k + 1, n_c - 1)
                _prefetch(erow0 + nk, nb)
                _drain(b)
                for g in range(G):
                    sl = pl.ds(g * _LANES, _LANES)
                    e_v[b, sl] = (_tanh_via_exp(a1g_v[b, sl] + a2g_v[b, sl])
                                  * ddg_v[b, sl] * dsg_v[b, sl])
                _drain_rows(b)

                @plsc.parallel_loop(0, _CHUNK, unroll=4)
                def _(i):
                    es = e_v[b, pl.ds(i, _LANES)][0]
                    for kk in range(D // (2 * _LANES)):
                        x32 = rows_v[b, i, pl.ds(kk * _LANES, _LANES)]
                        x = plsc.bitcast(x32, jnp.bfloat16)
                        pa, pb = plsc.unpack(
                            x, format=plsc.PackFormat.INTERLEAVED)
                        frows_v[i, pl.ds(kk * 2 * _LANES, _LANES)] = pa * es
                        frows_v[i, pl.ds(kk * 2 * _LANES + _LANES,
                                         _LANES)] = pb * es

                pltpu.sync_copy(frows_v, m_sh.at[dst_v.at[b]], add=True)

        _drain(0)
        _drain_rows(0)
        plsc.subcore_barrier()

        for r in range(RPT // _IDXW):
            pltpu.sync_copy(m_sh.at[pl.ds(base + r * _IDXW, _IDXW), :],
                            mm_hbm.at[cid, pl.ds(base + r * _IDXW, _IDXW), :])

    return sc_kernel


def _pick_bs(n):
    for cand in (1024, 1000, 512, 500, 256, 250, 128, 125, 64, 40, 32, 25,
                 16, 10, 8, 5, 4, 2, 1):
        if n % cand == 0:
            return cand
    return 1


def _gate_proj(h, w2, b2):
    n, d = h.shape
    bs = _pick_bs(n)

    def body(h_ref, w_ref, b_ref, o_ref):
        o_ref[...] = jnp.dot(h_ref[...], w_ref[...],
                             preferred_element_type=jnp.float32) + b_ref[...]

    return pl.pallas_call(
        body,
        grid=(n // bs,),
        in_specs=[pl.BlockSpec((bs, d), lambda i: (i, 0)),
                  pl.BlockSpec((d, 2), lambda i: (0, 0)),
                  pl.BlockSpec((1, 2), lambda i: (0, 0))],
        out_specs=pl.BlockSpec((bs, 2), lambda i: (i, 0)),
        out_shape=jax.ShapeDtypeStruct((n, 2), jnp.float32),
    )(h, w2, b2)


def _combine(h, mm):
    n, d = h.shape
    bs = _pick_bs(n)

    def body(h_ref, m0_ref, m1_ref, o_ref):
        o_ref[...] = jnp.maximum(
            _EPS * h_ref[...] + m0_ref[0] + m1_ref[0], 0.0)

    return pl.pallas_call(
        body,
        grid=(n // bs,),
        in_specs=[pl.BlockSpec((bs, d), lambda i: (i, 0)),
                  pl.BlockSpec((1, bs, d), lambda i: (0, i, 0)),
                  pl.BlockSpec((1, bs, d), lambda i: (1, i, 0))],
        out_specs=pl.BlockSpec((bs, d), lambda i: (i, 0)),
        out_shape=jax.ShapeDtypeStruct((n, d), jnp.float32),
    )(h, mm, mm)


_SC0_FRAC = 0.60


def kernel(h, edge_index, gate_w, gate_b):
    n, d = h.shape
    e = edge_index.shape[1]

    npad = -((n + 1) // -(_NS * _LANES)) * (_NS * _LANES)
    per_chunk = _NS * _CHUNK
    t = -(e // -(per_chunk * 16)) * 16
    n0 = int(round(t * _SC0_FRAC / 2)) * 2
    n0 = min(max(n0, 2), t - 2)
    n1 = t - n0
    e_pad = t * per_chunk

    src = edge_index[0]
    dst = edge_index[1]
    pad = e_pad - e
    srcp = jnp.concatenate(
        [src, jnp.zeros((pad,), jnp.int32)]).reshape(e_pad // _IDXW, _IDXW)
    dstp = jnp.concatenate(
        [dst, jnp.full((pad,), n, jnp.int32)]).reshape(e_pad // _IDXW, _IDXW)

    w_dst = gate_w[:d, 0]
    w_src = gate_w[d:, 0]
    w2 = jnp.stack([w_dst, w_src], axis=1)
    b2 = jnp.stack([gate_b[0], jnp.zeros((), jnp.float32)]).reshape(1, 2)

    a = _gate_proj(h, w2, b2)
    a1 = jnp.pad(a[:, 0], (0, npad - n))
    a2 = jnp.pad(a[:, 1], (0, npad - n))

    hbf = (h.reshape(n, d // 32, 2, 16).transpose(0, 1, 3, 2).reshape(n, d)
           .astype(jnp.bfloat16))
    hbf = jnp.pad(hbf, ((0, npad - n), (0, 0)))
    hbf = lax.bitcast_convert_type(hbf.reshape(npad, d // 2, 2), jnp.int32)

    mm = _make_sc_kernel(n, d, npad, n0, n1)(hbf, srcp, dstp, a1, a2)
    return _combine(h, mm)
